# baseline (speedup 1.0000x reference)
import jax
import jax.numpy as jnp
from jax.experimental import pallas as pl
from jax.experimental.pallas import tpu as pltpu

_PERIODS_PER_BLOCK = 4


def _add_kernel(x_ref, pos_ref, o_ref):
    P = pos_ref.shape[0]
    for k in range(_PERIODS_PER_BLOCK):
        sl = pl.ds(k * P, P)
        o_ref[sl, :] = x_ref[sl, :] + pos_ref[...]


def _tc_add(x2, pos_table):
    N, E = x2.shape
    P = pos_table.shape[0]
    R = _PERIODS_PER_BLOCK * P
    return pl.pallas_call(
        _add_kernel,
        grid=(N // R,),
        in_specs=[
            pl.BlockSpec((R, E), lambda b: (b, 0)),
            pl.BlockSpec((P, E), lambda b: (0, 0)),
        ],
        out_specs=pl.BlockSpec((R, E), lambda b: (b, 0)),
        out_shape=jax.ShapeDtypeStruct((N, E), x2.dtype),
        compiler_params=pltpu.CompilerParams(
            dimension_semantics=("parallel",),
        ),
    )(x2, pos_table)


def kernel(x, pos_table):
    B, P, E = x.shape
    x2 = x.reshape(B * P, E)
    split = 48 * P
    out_a = _tc_add(x2[:split], pos_table)
    out_b = _tc_add(x2[split:], pos_table)
    return jnp.concatenate([out_a, out_b], axis=0).reshape(B, P, E)
